# Initial kernel scaffold; baseline (speedup 1.0000x reference)
import jax
import jax.numpy as jnp
from jax.experimental import pallas as pl


def kernel(x, w_fused, b_fused):
    raise NotImplementedError("write your pallas kernel here")


# speedup vs baseline: 1.0116x; 1.0116x over previous
import functools

import jax
import jax.numpy as jnp
from jax.experimental import pallas as pl
from jax.experimental.pallas import tpu as pltpu


def _softmax_linear_block(x_ref, w_ref, b_ref, ones_ref, o_ref, *, c_out):
    logits = jnp.dot(x_ref[...], w_ref[...], preferred_element_type=jnp.float32)
    logits = logits + b_ref[...]
    m = jnp.max(logits, axis=-1, keepdims=True)
    e = jnp.exp(logits - m)
    denom = jnp.dot(e, ones_ref[...], preferred_element_type=jnp.float32)
    probs = e * pl.reciprocal(denom, approx=False)
    o_ref[...] = probs[:, :c_out]


@jax.jit
def kernel(x, w_fused, b_fused):
    B, D = x.shape
    C = w_fused.shape[1]
    CP = ((C + 127) // 128) * 128

    w_p = jnp.pad(w_fused.astype(jnp.float32), ((0, 0), (0, CP - C)))
    b_p = jnp.pad(jnp.reshape(b_fused, (1, -1)).astype(jnp.float32),
                  ((0, 0), (0, CP - C)), constant_values=-1e30)
    ones = jnp.ones((CP, CP), jnp.float32)

    block_b = min(2048, B)
    n_blocks = pl.cdiv(B, block_b)
    body = functools.partial(_softmax_linear_block, c_out=C)
    return pl.pallas_call(
        body,
        out_shape=jax.ShapeDtypeStruct((B, C), jnp.float32),
        grid=(n_blocks,),
        in_specs=[
            pl.BlockSpec((block_b, D), lambda i: (i, 0)),
            pl.BlockSpec((D, CP), lambda i: (0, 0)),
            pl.BlockSpec((1, CP), lambda i: (0, 0)),
            pl.BlockSpec((CP, CP), lambda i: (0, 0)),
        ],
        out_specs=pl.BlockSpec((block_b, C), lambda i: (i, 0)),
        compiler_params=pltpu.CompilerParams(
            dimension_semantics=("parallel",)),
    )(x, w_p, b_p, ones)


# speedup vs baseline: 2.1433x; 2.1186x over previous
import functools

import jax
import jax.numpy as jnp
from jax.experimental import pallas as pl
from jax.experimental.pallas import tpu as pltpu


def _softmax_linear_t_block(x_ref, wt_ref, bt_ref, ot_ref):
    logits_t = jax.lax.dot_general(
        wt_ref[...], x_ref[...],
        dimension_numbers=(((1,), (1,)), ((), ())),
        preferred_element_type=jnp.float32)
    logits_t = logits_t + bt_ref[...]
    m = jnp.max(logits_t, axis=0, keepdims=True)
    e = jnp.exp(logits_t - m)
    s = jnp.sum(e, axis=0, keepdims=True)
    ot_ref[...] = e * pl.reciprocal(s, approx=False)


@jax.jit
def kernel(x, w_fused, b_fused):
    B, D = x.shape
    C = w_fused.shape[1]

    block_b = min(2048, B)
    n_blocks = pl.cdiv(B, block_b)

    w_t = jnp.transpose(w_fused.astype(jnp.float32))
    b_t = jnp.broadcast_to(
        jnp.reshape(b_fused.astype(jnp.float32), (-1, 1)), (C, block_b))

    out_t = pl.pallas_call(
        _softmax_linear_t_block,
        out_shape=jax.ShapeDtypeStruct((C, B), jnp.float32),
        grid=(n_blocks,),
        in_specs=[
            pl.BlockSpec((block_b, D), lambda i: (i, 0)),
            pl.BlockSpec((C, D), lambda i: (0, 0)),
            pl.BlockSpec((C, block_b), lambda i: (0, 0)),
        ],
        out_specs=pl.BlockSpec((C, block_b), lambda i: (0, i)),
        compiler_params=pltpu.CompilerParams(
            dimension_semantics=("parallel",)),
    )(x, w_t, b_t)
    return jnp.transpose(out_t)


# speedup vs baseline: 3.6573x; 1.7064x over previous
import functools

import jax
import jax.numpy as jnp
from jax.experimental import pallas as pl
from jax.experimental.pallas import tpu as pltpu


def _softmax_linear_t_block(x_ref, wt_ref, bt_ref, ot_ref):
    logits_t = jax.lax.dot_general(
        wt_ref[...], x_ref[...],
        dimension_numbers=(((1,), (1,)), ((), ())),
        preferred_element_type=jnp.float32)
    logits_t = logits_t + bt_ref[...]
    m = jnp.max(logits_t, axis=0, keepdims=True)
    e = jnp.exp(logits_t - m)
    s = jnp.sum(e, axis=0, keepdims=True)
    ot_ref[...] = e * pl.reciprocal(s, approx=False)


@jax.jit
def kernel(x, w_fused, b_fused):
    B, D = x.shape
    C = w_fused.shape[1]

    block_b = min(8192, B)
    n_blocks = pl.cdiv(B, block_b)

    w_t = jnp.transpose(w_fused.astype(jnp.float32))
    b_t = jnp.broadcast_to(
        jnp.reshape(b_fused.astype(jnp.float32), (-1, 1)), (C, block_b))

    out_t = pl.pallas_call(
        _softmax_linear_t_block,
        out_shape=jax.ShapeDtypeStruct((C, B), jnp.float32),
        grid=(n_blocks,),
        in_specs=[
            pl.BlockSpec((block_b, D), lambda i: (i, 0)),
            pl.BlockSpec((C, D), lambda i: (0, 0)),
            pl.BlockSpec((C, block_b), lambda i: (0, 0)),
        ],
        out_specs=pl.BlockSpec((C, block_b), lambda i: (0, i)),
        compiler_params=pltpu.CompilerParams(
            dimension_semantics=("parallel",)),
    )(x, w_t, b_t)
    return jnp.transpose(out_t)


# speedup vs baseline: 3.7596x; 1.0280x over previous
import functools

import jax
import jax.numpy as jnp
from jax.experimental import pallas as pl
from jax.experimental.pallas import tpu as pltpu


def _softmax_linear_t_block(x_ref, wt_ref, bt_ref, ot_ref):
    logits_t = jax.lax.dot_general(
        wt_ref[...], x_ref[...],
        dimension_numbers=(((1,), (1,)), ((), ())),
        preferred_element_type=jnp.float32)
    logits_t = logits_t + bt_ref[...]
    m = jnp.max(logits_t, axis=0, keepdims=True)
    e = jnp.exp(logits_t - m)
    s = jnp.sum(e, axis=0, keepdims=True)
    ot_ref[...] = e * pl.reciprocal(s, approx=False)


@jax.jit
def kernel(x, w_fused, b_fused):
    B, D = x.shape
    C = w_fused.shape[1]

    block_b = min(16384, B)
    n_blocks = pl.cdiv(B, block_b)

    w_t = jnp.transpose(w_fused.astype(jnp.float32))
    b_t = jnp.broadcast_to(
        jnp.reshape(b_fused.astype(jnp.float32), (-1, 1)), (C, block_b))

    out_t = pl.pallas_call(
        _softmax_linear_t_block,
        out_shape=jax.ShapeDtypeStruct((C, B), jnp.float32),
        grid=(n_blocks,),
        in_specs=[
            pl.BlockSpec((block_b, D), lambda i: (i, 0)),
            pl.BlockSpec((C, D), lambda i: (0, 0)),
            pl.BlockSpec((C, block_b), lambda i: (0, 0)),
        ],
        out_specs=pl.BlockSpec((C, block_b), lambda i: (0, i)),
        compiler_params=pltpu.CompilerParams(
            dimension_semantics=("parallel",)),
    )(x, w_t, b_t)
    return jnp.transpose(out_t)
